# baseline (speedup 1.0000x reference)
import jax
import jax.numpy as jnp
from jax.experimental import pallas as pl
from jax.experimental.pallas import tpu as pltpu

V = 5000
K = 50
NFEAT = 128
HIDDEN = 128
B = 64
ALPHA = 0.1
ITERATE = 2
EPS = 1e-10

BM = 200
NBLK = V // BM


def _digamma(x):
    acc = jnp.zeros_like(x)
    for j in range(8):
        acc = acc + 1.0 / (x + float(j))
    y = x + 8.0
    inv = 1.0 / y
    inv2 = inv * inv
    series = (jnp.log(y) - 0.5 * inv
              - inv2 * (1.0 / 12.0 - inv2 * (1.0 / 120.0 - inv2 * (1.0 / 252.0))))
    return series - acc


def _fused_kernel(adj_ref, x_ref, w1_ref, w2_ref, bp_ref, kappa_ref, cnt_ref,
                  logbeta_ref, tphi_ref, xw1_s, m_s, z_s):
    p = pl.program_id(0)
    i = pl.program_id(1)

    @pl.when(jnp.logical_and(p == 0, i == 0))
    def _():
        xw1_s[...] = jnp.dot(x_ref[...], w1_ref[...],
                             preferred_element_type=jnp.float32)

    @pl.when(p == 0)
    def _():
        h1 = jnp.maximum(
            jnp.dot(adj_ref[...], xw1_s[...], preferred_element_type=jnp.float32),
            0.0)
        m_s[pl.ds(i * BM, BM), :] = jnp.dot(
            h1, w2_ref[...], preferred_element_type=jnp.float32)

    @pl.when(p == 1)
    def _():
        z_s[pl.ds(i * BM, BM), :] = jnp.dot(
            adj_ref[...], m_s[...], preferred_element_type=jnp.float32)

    @pl.when(jnp.logical_and(p == 1, i == NBLK - 1))
    def _():
        s = jax.nn.sigmoid(kappa_ref[...])
        beta = s * bp_ref[...] + (1.0 - s) * z_s[...].T
        mx = jnp.max(beta, axis=1, keepdims=True)
        lse = mx + jnp.log(jnp.sum(jnp.exp(beta - mx), axis=1, keepdims=True))
        logbeta = beta - lse
        logbeta_ref[...] = logbeta

        beta_e = jnp.exp(logbeta)
        cnt = cnt_ref[...]
        csum = jnp.sum(cnt, axis=1, keepdims=True)
        g0 = ALPHA + csum / K
        e0 = jnp.exp(_digamma(g0) - _digamma(K * g0))
        E = jnp.broadcast_to(e0, (B, K)).astype(jnp.float32)
        for _ in range(ITERATE):
            D = jnp.dot(E, beta_e, preferred_element_type=jnp.float32) + K * EPS
            r = cnt / D
            g = (ALPHA
                 + E * jax.lax.dot_general(
                     r, beta_e, (((1,), (1,)), ((), ())),
                     preferred_element_type=jnp.float32)
                 + EPS * jnp.sum(r, axis=1, keepdims=True))
            gs = jnp.sum(g, axis=1, keepdims=True)
            E = jnp.exp(_digamma(g) - _digamma(gs))
        D = jnp.dot(E, beta_e, preferred_element_type=jnp.float32) + K * EPS
        r = cnt / D
        tp = (beta_e * jax.lax.dot_general(
                  E, r, (((0,), (0,)), ((), ())),
                  preferred_element_type=jnp.float32)
              + EPS * jnp.sum(r, axis=0, keepdims=True))
        tphi_ref[...] = tp


def _run(adj, x, W1, W2, betat_param, kappa, cnt, interpret=False):
    return pl.pallas_call(
        _fused_kernel,
        grid=(2, NBLK),
        in_specs=[
            pl.BlockSpec((BM, V), lambda p, i: (i, 0)),
            pl.BlockSpec((V, NFEAT), lambda p, i: (0, 0)),
            pl.BlockSpec((NFEAT, HIDDEN), lambda p, i: (0, 0)),
            pl.BlockSpec((HIDDEN, K), lambda p, i: (0, 0)),
            pl.BlockSpec((K, V), lambda p, i: (0, 0)),
            pl.BlockSpec((K, 1), lambda p, i: (0, 0)),
            pl.BlockSpec((B, V), lambda p, i: (0, 0)),
        ],
        out_specs=[
            pl.BlockSpec((K, V), lambda p, i: (0, 0)),
            pl.BlockSpec((K, V), lambda p, i: (0, 0)),
        ],
        out_shape=[
            jax.ShapeDtypeStruct((K, V), jnp.float32),
            jax.ShapeDtypeStruct((K, V), jnp.float32),
        ],
        scratch_shapes=[
            pltpu.VMEM((V, NFEAT), jnp.float32),
            pltpu.VMEM((V, K), jnp.float32),
            pltpu.VMEM((V, K), jnp.float32),
        ],
        interpret=interpret,
    )(adj, x, W1, W2, betat_param, kappa, cnt)


def kernel(inputs, x, adj, weightgc1, weightgc2, betat, W1, W2, betat_param, kappa):
    cnt = inputs.astype(jnp.float32)
    logbeta, total_phi = _run(adj, x, W1, W2, betat_param, kappa, cnt)
    return (logbeta, total_phi)


# speedup vs baseline: 1.2206x; 1.2206x over previous
import jax
import jax.numpy as jnp
from jax.experimental import pallas as pl
from jax.experimental.pallas import tpu as pltpu

V = 5000
K = 50
NFEAT = 128
HIDDEN = 128
B = 64
ALPHA = 0.1
ITERATE = 2
EPS = 1e-10

BM = 1000
NBLK = V // BM


def _digamma(x):
    acc = jnp.zeros_like(x)
    for j in range(8):
        acc = acc + 1.0 / (x + float(j))
    y = x + 8.0
    inv = 1.0 / y
    inv2 = inv * inv
    series = (jnp.log(y) - 0.5 * inv
              - inv2 * (1.0 / 12.0 - inv2 * (1.0 / 120.0 - inv2 * (1.0 / 252.0))))
    return series - acc


def _fused_kernel(adj_ref, x_ref, w1_ref, w2_ref, bp_ref, kappa_ref, cnt_ref,
                  logbeta_ref, tphi_ref, xw1_s, h_s, m_s, z_s):
    p = pl.program_id(0)
    i = pl.program_id(1)

    @pl.when(jnp.logical_and(p == 0, i == 0))
    def _():
        xw1_s[...] = jnp.dot(x_ref[...], w1_ref[...],
                             preferred_element_type=jnp.float32)

    @pl.when(p == 0)
    def _():
        h_s[pl.ds(i * BM, BM), :] = jnp.dot(
            adj_ref[...], xw1_s[...], preferred_element_type=jnp.float32)

    @pl.when(jnp.logical_and(p == 1, i == 0))
    def _():
        m_s[...] = jnp.dot(jnp.maximum(h_s[...], 0.0), w2_ref[...],
                           preferred_element_type=jnp.float32)

    @pl.when(p == 1)
    def _():
        row = (NBLK - 1 - i) * BM
        z_s[pl.ds(row, BM), :] = jnp.dot(
            adj_ref[...], m_s[...], preferred_element_type=jnp.float32)

    @pl.when(jnp.logical_and(p == 1, i == NBLK - 1))
    def _():
        s = jax.nn.sigmoid(kappa_ref[...])
        beta = s * bp_ref[...] + (1.0 - s) * z_s[...].T
        mx = jnp.max(beta, axis=1, keepdims=True)
        lse = mx + jnp.log(jnp.sum(jnp.exp(beta - mx), axis=1, keepdims=True))
        logbeta = beta - lse
        logbeta_ref[...] = logbeta

        beta_e = jnp.exp(logbeta)
        cnt = cnt_ref[...]
        csum = jnp.sum(cnt, axis=1, keepdims=True)
        g0 = ALPHA + csum / K
        e0 = jnp.exp(_digamma(g0) - _digamma(K * g0))
        E = jnp.broadcast_to(e0, (B, K)).astype(jnp.float32)
        for _ in range(ITERATE):
            D = jnp.dot(E, beta_e, preferred_element_type=jnp.float32) + K * EPS
            r = cnt / D
            g = (ALPHA
                 + E * jax.lax.dot_general(
                     r, beta_e, (((1,), (1,)), ((), ())),
                     preferred_element_type=jnp.float32)
                 + EPS * jnp.sum(r, axis=1, keepdims=True))
            gs = jnp.sum(g, axis=1, keepdims=True)
            E = jnp.exp(_digamma(g) - _digamma(gs))
        D = jnp.dot(E, beta_e, preferred_element_type=jnp.float32) + K * EPS
        r = cnt / D
        tp = (beta_e * jax.lax.dot_general(
                  E, r, (((0,), (0,)), ((), ())),
                  preferred_element_type=jnp.float32)
              + EPS * jnp.sum(r, axis=0, keepdims=True))
        tphi_ref[...] = tp


def _adj_index(p, i):
    return (jnp.where(p == 0, i, NBLK - 1 - i), 0)


def _run(adj, x, W1, W2, betat_param, kappa, cnt, interpret=False):
    return pl.pallas_call(
        _fused_kernel,
        grid=(2, NBLK),
        in_specs=[
            pl.BlockSpec((BM, V), _adj_index),
            pl.BlockSpec((V, NFEAT), lambda p, i: (0, 0)),
            pl.BlockSpec((NFEAT, HIDDEN), lambda p, i: (0, 0)),
            pl.BlockSpec((HIDDEN, K), lambda p, i: (0, 0)),
            pl.BlockSpec((K, V), lambda p, i: (0, 0)),
            pl.BlockSpec((K, 1), lambda p, i: (0, 0)),
            pl.BlockSpec((B, V), lambda p, i: (0, 0)),
        ],
        out_specs=[
            pl.BlockSpec((K, V), lambda p, i: (0, 0)),
            pl.BlockSpec((K, V), lambda p, i: (0, 0)),
        ],
        out_shape=[
            jax.ShapeDtypeStruct((K, V), jnp.float32),
            jax.ShapeDtypeStruct((K, V), jnp.float32),
        ],
        scratch_shapes=[
            pltpu.VMEM((V, NFEAT), jnp.float32),
            pltpu.VMEM((V, NFEAT), jnp.float32),
            pltpu.VMEM((V, K), jnp.float32),
            pltpu.VMEM((V, K), jnp.float32),
        ],
        interpret=interpret,
    )(adj, x, W1, W2, betat_param, kappa, cnt)


def kernel(inputs, x, adj, weightgc1, weightgc2, betat, W1, W2, betat_param, kappa):
    cnt = inputs.astype(jnp.float32)
    logbeta, total_phi = _run(adj, x, W1, W2, betat_param, kappa, cnt)
    return (logbeta, total_phi)
